# baseline (speedup 1.0000x reference)
import math

import jax
import jax.numpy as jnp
from jax import lax
from jax.experimental import pallas as pl
from jax.experimental.pallas import tpu as pltpu
from jax.experimental.pallas import tpu_sc as plsc

_DIM = 64
_SCALE = math.sqrt(_DIM)
_NC = 2
_NS = 16
_NW = _NC * _NS
_CH = 512
_LANES = 16


def _make_kernel(n_rows: int):
    rows_per_w = n_rows // _NW
    n_chunks = rows_per_w // _CH
    mesh = plsc.VectorSubcoreMesh(core_axis_name="c", subcore_axis_name="s")

    def body(x_hbm, table_hbm, out_hbm, idx_v, rows_v, gsem):
        wid = lax.axis_index("s") * _NC + lax.axis_index("c")
        base = wid * rows_per_w
        pltpu.sync_copy(x_hbm.at[pl.ds(base, rows_per_w)], idx_v)

        @pl.loop(0, n_chunks)
        def _chunk(c):
            off = c * _CH
            pltpu.async_copy(
                table_hbm.at[idx_v.at[pl.ds(off, _CH)]], rows_v, gsem
            ).wait()

            @pl.loop(0, _CH)
            def _row(r):
                for j in range(_DIM // _LANES):
                    sl = pl.ds(j * _LANES, _LANES)
                    rows_v[r, sl] = rows_v[r, sl] * _SCALE

            pltpu.sync_copy(rows_v, out_hbm.at[pl.ds(base + off, _CH)])

    return pl.kernel(
        body,
        out_type=jax.ShapeDtypeStruct((n_rows, _DIM), jnp.float32),
        mesh=mesh,
        scratch_types=[
            pltpu.VMEM((rows_per_w,), jnp.int32),
            pltpu.VMEM((_CH, _DIM), jnp.float32),
            pltpu.SemaphoreType.DMA,
        ],
        compiler_params=pltpu.CompilerParams(use_tc_tiling_on_sc=False),
    )


def kernel(x, table):
    b, s = x.shape
    idx = x.reshape(-1).astype(jnp.int32)
    out = _make_kernel(idx.shape[0])(idx, table)
    return out.reshape(b, s, _DIM)


# speedup vs baseline: 1.1156x; 1.1156x over previous
import math

import jax
import jax.numpy as jnp
from jax import lax
from jax.experimental import pallas as pl
from jax.experimental.pallas import tpu as pltpu
from jax.experimental.pallas import tpu_sc as plsc

_DIM = 64
_SCALE = math.sqrt(_DIM)
_NC = 2
_NS = 16
_NW = _NC * _NS
_CH = 512
_LANES = 16
_UNROLL = 8


def _make_kernel(n_rows: int):
    rows_per_w = n_rows // _NW
    n_chunks = rows_per_w // _CH
    mesh = plsc.VectorSubcoreMesh(core_axis_name="c", subcore_axis_name="s")

    def body(x_hbm, table_hbm, out_hbm, idx_v, rows0, rows1, sem0, sem1):
        wid = lax.axis_index("s") * _NC + lax.axis_index("c")
        base = wid * rows_per_w
        pltpu.sync_copy(x_hbm.at[pl.ds(base, rows_per_w)], idx_v)

        bufs = (rows0, rows1)
        sems = (sem0, sem1)

        def gather(c, b):
            return pltpu.make_async_copy(
                table_hbm.at[idx_v.at[pl.ds(c * _CH, _CH)]], bufs[b], sems[b]
            )

        gather(0, 0).start()
        gather(1, 1).start()

        @pl.loop(0, n_chunks, step=2)
        def _ring(g):
            for b in range(2):
                c = g + b
                gather(c, b).wait()

                @pl.loop(0, _CH, step=_UNROLL)
                def _scale(r0):
                    for rr in range(_UNROLL):
                        for j in range(_DIM // _LANES):
                            sl = pl.ds(j * _LANES, _LANES)
                            bufs[b][r0 + rr, sl] = bufs[b][r0 + rr, sl] * _SCALE

                pltpu.sync_copy(bufs[b], out_hbm.at[pl.ds(base + c * _CH, _CH)])

                @pl.when(c < n_chunks - 2)
                def _():
                    gather(c + 2, b).start()

    return pl.kernel(
        body,
        out_type=jax.ShapeDtypeStruct((n_rows, _DIM), jnp.float32),
        mesh=mesh,
        scratch_types=[
            pltpu.VMEM((rows_per_w,), jnp.int32),
            pltpu.VMEM((_CH, _DIM), jnp.float32),
            pltpu.VMEM((_CH, _DIM), jnp.float32),
            pltpu.SemaphoreType.DMA,
            pltpu.SemaphoreType.DMA,
        ],
        compiler_params=pltpu.CompilerParams(use_tc_tiling_on_sc=False),
    )


def kernel(x, table):
    b, s = x.shape
    idx = x.reshape(-1).astype(jnp.int32)
    out = _make_kernel(idx.shape[0])(idx, table)
    return out.reshape(b, s, _DIM)
